# baseline (speedup 1.0000x reference)
import functools

import jax
import jax.numpy as jnp
from jax import lax
from jax.experimental import pallas as pl
from jax.experimental.pallas import tpu as pltpu
from jax.experimental.pallas import tpu_sc as plsc

B = 16384
F = 4096
L = 16

_info = plsc.get_sparse_core_info()
NC = _info.num_cores
NS = _info.num_subcores
NW = NC * NS

ROWS_PER_W = B // NW
RBLK = 8
NBLK = ROWS_PER_W // RBLK
NCHUNK = F // L

_mesh = plsc.VectorSubcoreMesh(core_axis_name="c", subcore_axis_name="s")


@functools.partial(
    pl.kernel,
    mesh=_mesh,
    compiler_params=pltpu.CompilerParams(needs_layout_passes=False),
    out_type=(
        jax.ShapeDtypeStruct((B * F,), jnp.float32),
        jax.ShapeDtypeStruct((B,), jnp.float32),
    ),
    scratch_types=[
        pltpu.VMEM((F,), jnp.int32),
        pltpu.VMEM((RBLK * F,), jnp.float32),
        pltpu.VMEM((RBLK * F,), jnp.float32),
        pltpu.VMEM((ROWS_PER_W,), jnp.float32),
    ],
)
def _permute(x_hbm, perm_hbm, out_hbm, ld_hbm, perm_v, inb, outb, ldb):
    wid = lax.axis_index("s") * NC + lax.axis_index("c")
    base = wid * ROWS_PER_W

    pltpu.sync_copy(perm_hbm, perm_v)

    zero = jnp.zeros((L,), jnp.float32)

    def zero_body(t, carry):
        ldb[pl.ds(t * L, L)] = zero
        return carry

    lax.fori_loop(0, ROWS_PER_W // L, zero_body, 0)
    pltpu.sync_copy(ldb, ld_hbm.at[pl.ds(base, ROWS_PER_W)])

    def block_body(g, carry):
        elem0 = (base + g * RBLK) * F
        pltpu.sync_copy(x_hbm.at[pl.ds(elem0, RBLK * F)], inb)

        def chunk_body(jc, c2):
            j0 = jc * L
            pc = perm_v[pl.ds(j0, L)]
            for r in range(RBLK):
                idx = pc + jnp.full((L,), r * F, jnp.int32)
                outb[pl.ds(r * F + j0, L)] = plsc.load_gather(inb, [idx])
            return c2

        lax.fori_loop(0, NCHUNK, chunk_body, 0)
        pltpu.sync_copy(outb, out_hbm.at[pl.ds(elem0, RBLK * F)])
        return carry

    lax.fori_loop(0, NBLK, block_body, 0)


def kernel(x, perm, inv_perm):
    del inv_perm
    out_flat, log_det = _permute(x.reshape(-1), perm.astype(jnp.int32))
    return (out_flat.reshape(B, F), log_det)


# speedup vs baseline: 2.0143x; 2.0143x over previous
import functools

import jax
import jax.numpy as jnp
from jax import lax
from jax.experimental import pallas as pl
from jax.experimental.pallas import tpu as pltpu
from jax.experimental.pallas import tpu_sc as plsc

B = 16384
F = 4096
L = 16

_info = plsc.get_sparse_core_info()
NC = _info.num_cores
NS = _info.num_subcores
NW = NC * NS

ROWS_PER_W = B // NW
RBLK = 4
NBUF = 2
NBLK = ROWS_PER_W // RBLK
NCHUNK = F // L

_mesh = plsc.VectorSubcoreMesh(core_axis_name="c", subcore_axis_name="s")


@functools.partial(
    pl.kernel,
    mesh=_mesh,
    compiler_params=pltpu.CompilerParams(needs_layout_passes=False),
    out_type=(
        jax.ShapeDtypeStruct((B * F,), jnp.float32),
        jax.ShapeDtypeStruct((B,), jnp.float32),
    ),
    scratch_types=[
        pltpu.VMEM((F,), jnp.int32),
        pltpu.VMEM((RBLK * F,), jnp.float32),
        pltpu.VMEM((RBLK * F,), jnp.float32),
        pltpu.VMEM((RBLK * F,), jnp.float32),
        pltpu.VMEM((RBLK * F,), jnp.float32),
        pltpu.VMEM((ROWS_PER_W,), jnp.float32),
        pltpu.SemaphoreType.DMA,
        pltpu.SemaphoreType.DMA,
        pltpu.SemaphoreType.DMA,
        pltpu.SemaphoreType.DMA,
    ],
)
def _permute(x_hbm, perm_hbm, out_hbm, ld_hbm, perm_v,
             inb0, inb1, outb0, outb1, ldb, is0, is1, os0, os1):
    wid = lax.axis_index("s") * NC + lax.axis_index("c")
    base = wid * ROWS_PER_W
    inbufs, outbufs = [inb0, inb1], [outb0, outb1]
    isems, osems = [is0, is1], [os0, os1]

    pltpu.sync_copy(perm_hbm, perm_v)

    zero = jnp.zeros((L,), jnp.float32)

    def zero_body(t, carry):
        ldb[pl.ds(t * L, L)] = zero
        return carry

    lax.fori_loop(0, ROWS_PER_W // L, zero_body, 0)
    pltpu.sync_copy(ldb, ld_hbm.at[pl.ds(base, ROWS_PER_W)])

    def in_slice(gb):
        return x_hbm.at[pl.ds((base + gb * RBLK) * F, RBLK * F)]

    def out_slice(gb):
        return out_hbm.at[pl.ds((base + gb * RBLK) * F, RBLK * F)]

    for b in range(NBUF):
        pltpu.async_copy(in_slice(b), inbufs[b], isems[b])

    def block_body(g, carry):
        for b in range(NBUF):
            gb = g * NBUF + b
            pltpu.make_async_copy(in_slice(gb), inbufs[b], isems[b]).wait()

            @pl.when(gb >= NBUF)
            def _wait_out():
                pltpu.make_async_copy(
                    outbufs[b], out_slice(gb), osems[b]).wait()

            inb, outb = inbufs[b], outbufs[b]

            @plsc.parallel_loop(0, NCHUNK, unroll=4)
            def chunk_body(jc):
                j0 = jc * L
                pc = perm_v[pl.ds(j0, L)]
                for r in range(RBLK):
                    idx = pc + jnp.full((L,), r * F, jnp.int32)
                    outb[pl.ds(r * F + j0, L)] = plsc.load_gather(inb, [idx])

            pltpu.async_copy(outb, out_slice(gb), osems[b])

            @pl.when(gb + NBUF < NBLK)
            def _next_in():
                pltpu.async_copy(in_slice(gb + NBUF), inbufs[b], isems[b])
        return carry

    lax.fori_loop(0, NBLK // NBUF, block_body, 0)

    for b in range(NBUF):
        pltpu.make_async_copy(
            outbufs[b], out_slice(NBLK - NBUF + b), osems[b]).wait()


def kernel(x, perm, inv_perm):
    del inv_perm
    out_flat, log_det = _permute(x.reshape(-1), perm.astype(jnp.int32))
    return (out_flat.reshape(B, F), log_det)
